# baseline (speedup 1.0000x reference)
import functools

import jax
import jax.numpy as jnp
from jax import lax
from jax.experimental import pallas as pl
from jax.experimental.pallas import tpu as pltpu
from jax.experimental.pallas import tpu_sc as plsc

_NC = 2
_NS = 16
_LANES = 16
_NW = _NC * _NS


def _make_pair_dot(batch: int, dim: int):
    assert batch % _NW == 0
    b_per_w = batch // _NW
    chunk = 128
    n_chunks = b_per_w // chunk
    n_groups = b_per_w // _LANES

    mesh = plsc.VectorSubcoreMesh(core_axis_name="c", subcore_axis_name="s")

    @functools.partial(
        pl.kernel,
        out_type=jax.ShapeDtypeStruct((batch,), jnp.float32),
        mesh=mesh,
        compiler_params=pltpu.CompilerParams(
            needs_layout_passes=False, use_tc_tiling_on_sc=False),
        scratch_types=[
            pltpu.VMEM((b_per_w,), jnp.int32),
            pltpu.VMEM((b_per_w,), jnp.int32),
            pltpu.VMEM((b_per_w, dim), jnp.float32),
            pltpu.VMEM((b_per_w, dim), jnp.float32),
            pltpu.VMEM((b_per_w,), jnp.float32),
            pltpu.SemaphoreType.DMA,
        ],
    )
    def pair_dot(i_hbm, j_hbm, table_hbm, out_hbm,
                 idx_i, idx_j, rows_i, rows_j, res, sem):
        wid = lax.axis_index("s") * _NC + lax.axis_index("c")
        base = wid * b_per_w

        pltpu.sync_copy(i_hbm.at[pl.ds(base, b_per_w)], idx_i)
        pltpu.sync_copy(j_hbm.at[pl.ds(base, b_per_w)], idx_j)

        copies = []
        for c in range(n_chunks):
            sl = pl.ds(c * chunk, chunk)
            copies.append(
                pltpu.async_copy(table_hbm.at[idx_i.at[sl]], rows_i.at[sl], sem))
            copies.append(
                pltpu.async_copy(table_hbm.at[idx_j.at[sl]], rows_j.at[sl], sem))
        for cp in copies:
            cp.wait()

        lanes = lax.iota(jnp.int32, _LANES)

        def group_body(g, carry):
            rows = g * _LANES + lanes
            acc = jnp.zeros((_LANES,), jnp.float32)
            for d in range(dim):
                col = jnp.full((_LANES,), d, jnp.int32)
                vi = plsc.load_gather(rows_i, [rows, col])
                vj = plsc.load_gather(rows_j, [rows, col])
                acc = acc + vi * vj
            res[pl.ds(g * _LANES, _LANES)] = acc
            return carry

        lax.fori_loop(0, n_groups, group_body, 0)

        pltpu.sync_copy(res, out_hbm.at[pl.ds(base, b_per_w)])

    return pair_dot


def kernel(pair, table):
    batch = pair.shape[0]
    dim = table.shape[1]
    i = pair[:, 0].astype(jnp.int32)
    j = pair[:, 1].astype(jnp.int32)
    sim = _make_pair_dot(batch, dim)(i, j, table)
    return sim[:, None]
